# baseline (speedup 1.0000x reference)
import jax
import jax.numpy as jnp
from jax import lax
from jax.experimental import pallas as pl
from jax.experimental.pallas import tpu as pltpu


_LANE = 128
_TP = 512
_S_CHUNK = 4


def _ceil_to(x: int, m: int) -> int:
    return ((x + m - 1) // m) * m


def _make_body(*, N_pad, TP, S, S_chunk, CPAD, r2):
    f32 = jnp.float32
    bf16 = jnp.bfloat16
    n_chunks = S // S_chunk
    NB = N_pad // _LANE
    dn = (((1,), (1,)), ((), ()))
    dnT = (((1,), (0,)), ((), ()))

    def body(new_pc_ref, xyz_ref, src_ref, corr_ref, ltri_ref, mblk_ref,
             mex_ref, out_ref):
        xyzf = xyz_ref[0].astype(f32)
        new_pc = new_pc_ref[0].astype(f32)
        src = src_ref[0]
        corrf = corr_ref[0]
        ltri = ltri_ref[...]
        mblk = mblk_ref[...]
        mex = mex_ref[...]

        d2 = None
        for d in range(3):
            diff = new_pc[:, d:d + 1] - xyzf[d:d + 1, :]
            sq = diff * diff
            d2 = sq if d2 is None else d2 + sq
        one = jnp.asarray(1.0, bf16)
        zero = jnp.asarray(0.0, bf16)
        f8 = jnp.float8_e4m3fn
        ibf = jnp.where(d2 < r2, 1.0, 0.0).astype(f8)

        intra_parts = [
            lax.dot_general(ibf[:, b * _LANE:(b + 1) * _LANE], ltri, dnT,
                            preferred_element_type=f32)
            for b in range(NB)
        ]
        intra = jnp.concatenate(intra_parts, axis=1)
        s_blk = lax.dot_general(ibf, mblk, dnT,
                                preferred_element_type=f32)
        offs = lax.dot_general(s_blk.astype(bf16), mex, dnT,
                               preferred_element_type=f32)
        rank = intra + offs
        count = jnp.sum(s_blk, axis=1, keepdims=True)

        rank_c = jnp.where(d2 < r2, jnp.minimum(rank, 64.0), -1.0).astype(bf16)

        lane_iota = lax.broadcasted_iota(jnp.int32, (1, N_pad), 1)
        first_oh = jnp.where((rank == 0.0) & (d2 < r2), 1.0, 0.0)
        fb = jnp.where(count > 0.0, first_oh,
                       jnp.where(lane_iota == 0, 1.0, 0.0)).astype(bf16)
        fbg = lax.dot_general(src, fb, dn, preferred_element_type=f32)

        count_row = jnp.transpose(count)
        row_iota = lax.broadcasted_iota(jnp.int32, (CPAD, TP), 0)
        xyz_rows = row_iota < 3

        def chunk_body(ci, carry):
            s0 = ci * S_chunk
            onehots = [
                jnp.where(rank_c == (s0 + j).astype(bf16), one, zero)
                for j in range(S_chunk)
            ]
            G = onehots[0] if S_chunk == 1 else jnp.concatenate(onehots, axis=0)
            G = G.astype(jnp.float8_e4m3fn)
            g = lax.dot_general(src, G, dn, preferred_element_type=f32)
            for j in range(S_chunk):
                s_f = (s0 + j).astype(f32)
                h = g[:, j * TP:(j + 1) * TP]
                h = h + jnp.where(count_row <= s_f, fbg, 0.0)
                out = h + jnp.where(xyz_rows, pltpu.roll(h, 5, 0), 0.0) - corrf
                row0 = pl.multiple_of((s0 + j) * CPAD, CPAD)
                out_ref[0, pl.ds(row0, CPAD), :] = out
            return carry

        unroll = True if n_chunks <= 4 else 2
        lax.fori_loop(0, n_chunks, chunk_body, 0, unroll=unroll)

    return body


def kernel(xyz, new_xyz, features):
    f32 = jnp.float32
    bf16 = jnp.bfloat16
    B, N, _ = xyz.shape
    P = new_xyz.shape[1]
    C = features.shape[1]
    S = 32
    radius = 0.2
    r2 = float(radius) * float(radius)
    out_c = 3 + C
    CPAD = _ceil_to(out_c + 3, 8)

    N_pad = _ceil_to(max(N, 1), _LANE)
    P_pad = _ceil_to(max(P, 1), _LANE)
    TP = _TP if P_pad % _TP == 0 else _LANE
    n_ptiles = P_pad // TP
    NB = N_pad // _LANE

    xyz_cf = jnp.swapaxes(xyz, 1, 2).astype(f32)
    xyz_hi = xyz_cf.astype(bf16)
    xyz_lo = (xyz_cf - xyz_hi.astype(f32)).astype(bf16)
    feat_hi = features.astype(f32).astype(bf16)
    pad_rows = CPAD - (out_c + 3)
    parts = [xyz_hi, feat_hi, xyz_lo]
    if pad_rows:
        parts.append(jnp.zeros((B, pad_rows, N), bf16))
    src = jnp.concatenate(parts, axis=1)

    corr = jnp.concatenate(
        [jnp.swapaxes(new_xyz, 1, 2).astype(f32),
         jnp.zeros((B, CPAD - 3, P), f32)], axis=1)

    new_pc = new_xyz.astype(f32)

    if N_pad > N:
        xyz_cf = jnp.concatenate(
            [xyz_cf, jnp.full((B, 3, N_pad - N), 1e18, f32)], axis=2)
        src = jnp.concatenate(
            [src, jnp.zeros((B, CPAD, N_pad - N), bf16)], axis=2)
    if P_pad > P:
        new_pc = jnp.concatenate(
            [new_pc, jnp.zeros((B, P_pad - P, 3), f32)], axis=1)
        corr = jnp.concatenate(
            [corr, jnp.zeros((B, CPAD, P_pad - P), f32)], axis=2)

    i128 = jnp.arange(_LANE)
    iN = jnp.arange(N_pad)
    f8 = jnp.float8_e4m3fn
    ltri = (i128[:, None] < i128[None, :]).astype(f8)
    mblk = ((iN[:, None] // _LANE) == i128[None, :]).astype(f8)
    mex = (i128[:, None] < (iN[None, :] // _LANE)).astype(f8)

    S_chunk = _S_CHUNK if S % _S_CHUNK == 0 else 1

    body = _make_body(N_pad=N_pad, TP=TP, S=S, S_chunk=S_chunk,
                      CPAD=CPAD, r2=r2)

    gather_flops = 2.0 * B * P_pad * (S + 1) * CPAD * N_pad
    rank_flops = 2.0 * B * P_pad * N_pad * (_LANE + 2.0)
    bytes_accessed = 4.0 * B * (3 + CPAD) * N_pad + 4.0 * B * S * CPAD * P_pad

    out3 = pl.pallas_call(
        body,
        out_shape=jax.ShapeDtypeStruct((B, S * CPAD, P_pad), f32),
        grid_spec=pltpu.PrefetchScalarGridSpec(
            num_scalar_prefetch=0,
            grid=(B, n_ptiles),
            in_specs=[
                pl.BlockSpec((1, TP, 3), lambda b, pt: (b, pt, 0)),
                pl.BlockSpec((1, 3, N_pad), lambda b, pt: (b, 0, 0)),
                pl.BlockSpec((1, CPAD, N_pad), lambda b, pt: (b, 0, 0)),
                pl.BlockSpec((1, CPAD, TP), lambda b, pt: (b, 0, pt)),
                pl.BlockSpec((_LANE, _LANE), lambda b, pt: (0, 0)),
                pl.BlockSpec((N_pad, _LANE), lambda b, pt: (0, 0)),
                pl.BlockSpec((_LANE, N_pad), lambda b, pt: (0, 0)),
            ],
            out_specs=pl.BlockSpec((1, S * CPAD, TP), lambda b, pt: (b, 0, pt)),
        ),
        compiler_params=pltpu.CompilerParams(
            dimension_semantics=("parallel", "parallel"),
            vmem_limit_bytes=56 * 1024 * 1024,
        ),
        cost_estimate=pl.CostEstimate(
            flops=int(gather_flops + rank_flops),
            transcendentals=0,
            bytes_accessed=int(bytes_accessed),
        ),
    )(new_pc, xyz_cf, src, corr, ltri, mblk, mex)

    out4 = out3.reshape(B, S, CPAD, P_pad)[:, :, :out_c, :P]
    return jnp.transpose(out4, (0, 2, 3, 1))


# speedup vs baseline: 1.2051x; 1.2051x over previous
import jax
import jax.numpy as jnp
from jax import lax
from jax.experimental import pallas as pl
from jax.experimental.pallas import tpu as pltpu


_LANE = 128
_TP = 512
_S_CHUNK = 8


def _ceil_to(x: int, m: int) -> int:
    return ((x + m - 1) // m) * m


def _make_body(*, N_pad, TP, S, S_chunk, CPAD, r2):
    f32 = jnp.float32
    bf16 = jnp.bfloat16
    n_chunks = S // S_chunk
    NB = N_pad // _LANE
    dn = (((1,), (1,)), ((), ()))
    dnT = (((1,), (0,)), ((), ()))

    def body(new_pc_ref, xyz_ref, src_ref, corr_ref, ltri_ref, mblk_ref,
             mex_ref, out_ref):
        xyzf = xyz_ref[0].astype(f32)
        new_pc = new_pc_ref[0].astype(f32)
        src = src_ref[0]
        corrf = corr_ref[0]
        ltri = ltri_ref[...]
        mblk = mblk_ref[...]
        mex = mex_ref[...]

        d2 = None
        for d in range(3):
            diff = new_pc[:, d:d + 1] - xyzf[d:d + 1, :]
            sq = diff * diff
            d2 = sq if d2 is None else d2 + sq
        one = jnp.asarray(1.0, bf16)
        zero = jnp.asarray(0.0, bf16)
        f8 = jnp.float8_e4m3fn
        ibf = jnp.where(d2 < r2, 1.0, 0.0).astype(f8)

        intra_parts = [
            lax.dot_general(ibf[:, b * _LANE:(b + 1) * _LANE], ltri, dnT,
                            preferred_element_type=f32)
            for b in range(NB)
        ]
        intra = jnp.concatenate(intra_parts, axis=1)
        s_blk = lax.dot_general(ibf, mblk, dnT,
                                preferred_element_type=f32)
        offs = lax.dot_general(s_blk.astype(bf16), mex, dnT,
                               preferred_element_type=f32)
        rank = intra + offs
        count = jnp.sum(s_blk, axis=1, keepdims=True)

        rank_c = jnp.where(d2 < r2, jnp.minimum(rank, 64.0), -1.0).astype(bf16)

        lane_iota = lax.broadcasted_iota(jnp.int32, (1, N_pad), 1)
        first_oh = jnp.where((rank == 0.0) & (d2 < r2), 1.0, 0.0)
        fb = jnp.where(count > 0.0, first_oh,
                       jnp.where(lane_iota == 0, 1.0, 0.0)).astype(bf16)
        fbg = lax.dot_general(src, fb, dn, preferred_element_type=f32)

        count_row = jnp.transpose(count)
        row_iota = lax.broadcasted_iota(jnp.int32, (CPAD, TP), 0)
        xyz_rows = row_iota < 3

        def chunk_body(ci, carry):
            s0 = ci * S_chunk
            onehots = [
                jnp.where(rank_c == (s0 + j).astype(bf16), one, zero)
                for j in range(S_chunk)
            ]
            G = onehots[0] if S_chunk == 1 else jnp.concatenate(onehots, axis=0)
            G = G.astype(jnp.float8_e4m3fn)
            g = lax.dot_general(src, G, dn, preferred_element_type=f32)
            for j in range(S_chunk):
                s_f = (s0 + j).astype(f32)
                h = g[:, j * TP:(j + 1) * TP]
                h = h + jnp.where(count_row <= s_f, fbg, 0.0)
                out = h + jnp.where(xyz_rows, pltpu.roll(h, 5, 0), 0.0) - corrf
                row0 = pl.multiple_of((s0 + j) * CPAD, CPAD)
                out_ref[0, pl.ds(row0, CPAD), :] = out
            return carry

        unroll = True if n_chunks <= 4 else 2
        lax.fori_loop(0, n_chunks, chunk_body, 0, unroll=unroll)

    return body


def kernel(xyz, new_xyz, features):
    f32 = jnp.float32
    bf16 = jnp.bfloat16
    B, N, _ = xyz.shape
    P = new_xyz.shape[1]
    C = features.shape[1]
    S = 32
    radius = 0.2
    r2 = float(radius) * float(radius)
    out_c = 3 + C
    CPAD = _ceil_to(out_c + 3, 8)

    N_pad = _ceil_to(max(N, 1), _LANE)
    P_pad = _ceil_to(max(P, 1), _LANE)
    TP = _TP if P_pad % _TP == 0 else _LANE
    n_ptiles = P_pad // TP
    NB = N_pad // _LANE

    xyz_cf = jnp.swapaxes(xyz, 1, 2).astype(f32)
    xyz_hi = xyz_cf.astype(bf16)
    xyz_lo = (xyz_cf - xyz_hi.astype(f32)).astype(bf16)
    feat_hi = features.astype(f32).astype(bf16)
    pad_rows = CPAD - (out_c + 3)
    parts = [xyz_hi, feat_hi, xyz_lo]
    if pad_rows:
        parts.append(jnp.zeros((B, pad_rows, N), bf16))
    src = jnp.concatenate(parts, axis=1)

    corr = jnp.concatenate(
        [jnp.swapaxes(new_xyz, 1, 2).astype(f32),
         jnp.zeros((B, CPAD - 3, P), f32)], axis=1)

    new_pc = new_xyz.astype(f32)

    if N_pad > N:
        xyz_cf = jnp.concatenate(
            [xyz_cf, jnp.full((B, 3, N_pad - N), 1e18, f32)], axis=2)
        src = jnp.concatenate(
            [src, jnp.zeros((B, CPAD, N_pad - N), bf16)], axis=2)
    if P_pad > P:
        new_pc = jnp.concatenate(
            [new_pc, jnp.zeros((B, P_pad - P, 3), f32)], axis=1)
        corr = jnp.concatenate(
            [corr, jnp.zeros((B, CPAD, P_pad - P), f32)], axis=2)

    i128 = jnp.arange(_LANE)
    iN = jnp.arange(N_pad)
    f8 = jnp.float8_e4m3fn
    ltri = (i128[:, None] < i128[None, :]).astype(f8)
    mblk = ((iN[:, None] // _LANE) == i128[None, :]).astype(f8)
    mex = (i128[:, None] < (iN[None, :] // _LANE)).astype(f8)

    S_chunk = _S_CHUNK if S % _S_CHUNK == 0 else 1

    body = _make_body(N_pad=N_pad, TP=TP, S=S, S_chunk=S_chunk,
                      CPAD=CPAD, r2=r2)

    gather_flops = 2.0 * B * P_pad * (S + 1) * CPAD * N_pad
    rank_flops = 2.0 * B * P_pad * N_pad * (_LANE + 2.0)
    bytes_accessed = 4.0 * B * (3 + CPAD) * N_pad + 4.0 * B * S * CPAD * P_pad

    out3 = pl.pallas_call(
        body,
        out_shape=jax.ShapeDtypeStruct((B, S * CPAD, P_pad), f32),
        grid_spec=pltpu.PrefetchScalarGridSpec(
            num_scalar_prefetch=0,
            grid=(B, n_ptiles),
            in_specs=[
                pl.BlockSpec((1, TP, 3), lambda b, pt: (b, pt, 0)),
                pl.BlockSpec((1, 3, N_pad), lambda b, pt: (b, 0, 0)),
                pl.BlockSpec((1, CPAD, N_pad), lambda b, pt: (b, 0, 0)),
                pl.BlockSpec((1, CPAD, TP), lambda b, pt: (b, 0, pt)),
                pl.BlockSpec((_LANE, _LANE), lambda b, pt: (0, 0)),
                pl.BlockSpec((N_pad, _LANE), lambda b, pt: (0, 0)),
                pl.BlockSpec((_LANE, N_pad), lambda b, pt: (0, 0)),
            ],
            out_specs=pl.BlockSpec((1, S * CPAD, TP), lambda b, pt: (b, 0, pt)),
        ),
        compiler_params=pltpu.CompilerParams(
            dimension_semantics=("parallel", "parallel"),
            vmem_limit_bytes=56 * 1024 * 1024,
        ),
        cost_estimate=pl.CostEstimate(
            flops=int(gather_flops + rank_flops),
            transcendentals=0,
            bytes_accessed=int(bytes_accessed),
        ),
    )(new_pc, xyz_cf, src, corr, ltri, mblk, mex)

    out4 = out3.reshape(B, S, CPAD, P_pad)[:, :, :out_c, :P]
    return jnp.transpose(out4, (0, 2, 3, 1))


# speedup vs baseline: 1.2078x; 1.0022x over previous
import jax
import jax.numpy as jnp
from jax import lax
from jax.experimental import pallas as pl
from jax.experimental.pallas import tpu as pltpu


_LANE = 128
_TP = 512
_S_CHUNK = 8


def _ceil_to(x: int, m: int) -> int:
    return ((x + m - 1) // m) * m


def _make_body(*, N_pad, TP, S, S_chunk, CPAD, r2):
    f32 = jnp.float32
    bf16 = jnp.bfloat16
    n_chunks = S // S_chunk
    NB = N_pad // _LANE
    dn = (((1,), (1,)), ((), ()))
    dnT = (((1,), (0,)), ((), ()))

    def body(new_pc_ref, xyz_ref, src_ref, corr_ref, ltri_ref, mblk_ref,
             mex_ref, out_ref):
        xyzf = xyz_ref[0].astype(f32)
        new_pc = new_pc_ref[0].astype(f32)
        src = src_ref[0]
        corrf = corr_ref[0]
        ltri = ltri_ref[...]
        mblk = mblk_ref[...]
        mex = mex_ref[...]

        d2 = None
        for d in range(3):
            diff = new_pc[:, d:d + 1] - xyzf[d:d + 1, :]
            sq = diff * diff
            d2 = sq if d2 is None else d2 + sq
        one = jnp.asarray(1.0, bf16)
        zero = jnp.asarray(0.0, bf16)
        f8 = jnp.float8_e4m3fn
        ibf = jnp.where(d2 < r2, 1.0, 0.0).astype(f8)

        intra_parts = [
            lax.dot_general(ibf[:, b * _LANE:(b + 1) * _LANE], ltri, dnT,
                            preferred_element_type=f32)
            for b in range(NB)
        ]
        intra = jnp.concatenate(intra_parts, axis=1)
        s_blk = lax.dot_general(ibf, mblk, dnT,
                                preferred_element_type=f32)
        offs = lax.dot_general(s_blk.astype(bf16), mex, dnT,
                               preferred_element_type=f32)
        rank = intra + offs
        count = jnp.sum(s_blk, axis=1, keepdims=True)

        rank_c = jnp.where(d2 < r2, jnp.minimum(rank, 64.0), -1.0).astype(bf16)

        lane_iota = lax.broadcasted_iota(jnp.int32, (1, N_pad), 1)
        first_oh = jnp.where((rank == 0.0) & (d2 < r2), 1.0, 0.0)
        fb = jnp.where(count > 0.0, first_oh,
                       jnp.where(lane_iota == 0, 1.0, 0.0)).astype(bf16)
        fbg = lax.dot_general(src, fb, dn, preferred_element_type=f32)

        count_row = jnp.transpose(count)
        row_iota = lax.broadcasted_iota(jnp.int32, (CPAD, TP), 0)
        xyz_rows = row_iota < 3

        def chunk_body(ci, carry):
            s0 = ci * S_chunk
            onehots = [
                jnp.where(rank_c == (s0 + j).astype(bf16), one, zero)
                for j in range(S_chunk)
            ]
            G = onehots[0] if S_chunk == 1 else jnp.concatenate(onehots, axis=0)
            G = G.astype(jnp.float8_e4m3fn)
            g = lax.dot_general(src, G, dn, preferred_element_type=f32)
            for j in range(S_chunk):
                s_f = (s0 + j).astype(f32)
                h = g[:, j * TP:(j + 1) * TP]
                h = h + jnp.where(count_row <= s_f, fbg, 0.0)
                out = h + jnp.where(xyz_rows, pltpu.roll(h, 5, 0), 0.0) - corrf
                row0 = pl.multiple_of((s0 + j) * CPAD, CPAD)
                out_ref[0, pl.ds(row0, CPAD), :] = out
            return carry

        unroll = True if n_chunks <= 4 else 2
        lax.fori_loop(0, n_chunks, chunk_body, 0, unroll=unroll)

    return body


def kernel(xyz, new_xyz, features):
    f32 = jnp.float32
    bf16 = jnp.bfloat16
    B, N, _ = xyz.shape
    P = new_xyz.shape[1]
    C = features.shape[1]
    S = 32
    radius = 0.2
    r2 = float(radius) * float(radius)
    out_c = 3 + C
    CPAD = _ceil_to(out_c + 3, 8)

    N_pad = _ceil_to(max(N, 1), _LANE)
    P_pad = _ceil_to(max(P, 1), _LANE)
    TP = _TP if P_pad % _TP == 0 else _LANE
    n_ptiles = P_pad // TP
    NB = N_pad // _LANE

    xyz_cf = jnp.swapaxes(xyz, 1, 2).astype(f32)
    xyz_hi = xyz_cf.astype(bf16)
    xyz_lo = (xyz_cf - xyz_hi.astype(f32)).astype(bf16)
    feat_hi = features.astype(f32).astype(bf16)
    pad_rows = CPAD - (out_c + 3)
    parts = [xyz_hi, feat_hi, xyz_lo]
    if pad_rows:
        parts.append(jnp.zeros((B, pad_rows, N), bf16))
    src = jnp.concatenate(parts, axis=1)

    corr = jnp.concatenate(
        [jnp.swapaxes(new_xyz, 1, 2).astype(f32),
         jnp.zeros((B, CPAD - 3, P), f32)], axis=1)

    new_pc = new_xyz.astype(f32)

    if N_pad > N:
        xyz_cf = jnp.concatenate(
            [xyz_cf, jnp.full((B, 3, N_pad - N), 1e18, f32)], axis=2)
        src = jnp.concatenate(
            [src, jnp.zeros((B, CPAD, N_pad - N), bf16)], axis=2)
    if P_pad > P:
        new_pc = jnp.concatenate(
            [new_pc, jnp.zeros((B, P_pad - P, 3), f32)], axis=1)
        corr = jnp.concatenate(
            [corr, jnp.zeros((B, CPAD, P_pad - P), f32)], axis=2)

    i128 = jnp.arange(_LANE)
    iN = jnp.arange(N_pad)
    f8 = jnp.float8_e4m3fn
    ltri = (i128[:, None] < i128[None, :]).astype(f8)
    mblk = ((iN[:, None] // _LANE) == i128[None, :]).astype(f8)
    mex = (i128[:, None] < (iN[None, :] // _LANE)).astype(f8)

    S_chunk = _S_CHUNK if S % _S_CHUNK == 0 else 1

    body = _make_body(N_pad=N_pad, TP=TP, S=S, S_chunk=S_chunk,
                      CPAD=CPAD, r2=r2)

    gather_flops = 2.0 * B * P_pad * (S + 1) * CPAD * N_pad
    rank_flops = 2.0 * B * P_pad * N_pad * (_LANE + 2.0)
    bytes_accessed = 4.0 * B * (3 + CPAD) * N_pad + 4.0 * B * S * CPAD * P_pad

    out3 = pl.pallas_call(
        body,
        out_shape=jax.ShapeDtypeStruct((B, S * CPAD, P_pad), f32),
        grid_spec=pltpu.PrefetchScalarGridSpec(
            num_scalar_prefetch=0,
            grid=(B, n_ptiles),
            in_specs=[
                pl.BlockSpec((1, TP, 3), lambda b, pt: (b, pt, 0)),
                pl.BlockSpec((1, 3, N_pad), lambda b, pt: (b, 0, 0)),
                pl.BlockSpec((1, CPAD, N_pad), lambda b, pt: (b, 0, 0)),
                pl.BlockSpec((1, CPAD, TP), lambda b, pt: (b, 0, pt)),
                pl.BlockSpec((_LANE, _LANE), lambda b, pt: (0, 0)),
                pl.BlockSpec((N_pad, _LANE), lambda b, pt: (0, 0)),
                pl.BlockSpec((_LANE, N_pad), lambda b, pt: (0, 0)),
            ],
            out_specs=pl.BlockSpec((1, S * CPAD, TP), lambda b, pt: (b, 0, pt)),
        ),
        compiler_params=pltpu.CompilerParams(
            dimension_semantics=("parallel", "parallel"),
            vmem_limit_bytes=64 * 1024 * 1024,
        ),
        cost_estimate=pl.CostEstimate(
            flops=int(gather_flops + rank_flops),
            transcendentals=0,
            bytes_accessed=int(bytes_accessed),
        ),
    )(new_pc, xyz_cf, src, corr, ltri, mblk, mex)

    out4 = out3.reshape(B, S, CPAD, P_pad)[:, :, :out_c, :P]
    return jnp.transpose(out4, (0, 2, 3, 1))
